# baseline (speedup 1.0000x reference)
import jax
import jax.numpy as jnp
from jax import lax
from jax.experimental import pallas as pl
from jax.experimental.pallas import tpu as pltpu
from jax.experimental.pallas import tpu_sc as plsc

N = 50000
D = 128
E = 400000

NC = 2
NS = 16
LANES = 16

ROWS_PER_TILE = 400
R = NS * ROWS_PER_TILE
NPASS = 4
N_PAD = NC * R * NPASS
K = 4
SEG = 1792
NSEG = 14
EPT = SEG * NSEG
E_PAD = EPT * NS
SBLK = SEG // LANES
CH = 128
DUMP = R
ROWBLK = 2000
NRB = N // ROWBLK


def _proj_body(xu_in, xi_in, wt, b, xu_out, xi_out):
    bb = b[...]
    xu_out[...] = jnp.dot(xu_in[...], wt[...],
                          preferred_element_type=jnp.float32) + bb
    xi_out[...] = jnp.dot(xi_in[...], wt[...],
                          preferred_element_type=jnp.float32) + bb


def _project(x_user, x_item, wt, b):
    return pl.pallas_call(
        _proj_body,
        grid=(NRB,),
        in_specs=[
            pl.BlockSpec((ROWBLK, D), lambda i: (i, 0)),
            pl.BlockSpec((ROWBLK, D), lambda i: (i, 0)),
            pl.BlockSpec((D, D), lambda i: (0, 0)),
            pl.BlockSpec((1, D), lambda i: (0, 0)),
        ],
        out_specs=[
            pl.BlockSpec((ROWBLK, D), lambda i: (i, 0)),
            pl.BlockSpec((ROWBLK, D), lambda i: (i, 0)),
        ],
        out_shape=[jax.ShapeDtypeStruct((N, D), jnp.float32)] * 2,
    )(x_user, x_item, wt, b)


def _sc_body(xu, xi, src_ub, dst_ub, src_ib, dst_ib,
             s_ub_out, deg_ub_out, s_ib_out, deg_ib_out,
             seg_src, seg_dst, gidx, sidx, rows4, zbuf, ones_v,
             dz, deg_stage, accum, dega, sem):
    c = lax.axis_index("c")
    s = lax.axis_index("s")
    iota = lax.iota(jnp.int32, LANES)
    z16f = jnp.zeros((LANES,), jnp.float32)
    z16i = jnp.zeros((LANES,), jnp.int32)
    dump16 = jnp.full((LANES,), DUMP, jnp.int32)

    for r_ in range(16):
        for cb in range(D // LANES):
            zbuf[r_, pl.ds(cb * LANES, LANES)] = z16f
    for cb in range(CH // LANES):
        ones_v[pl.ds(cb * LANES, LANES)] = jnp.ones((LANES,), jnp.float32)
    dz[pl.ds(0, LANES)] = z16f

    for (srch, dsth, table, s_out, d_out) in (
            (src_ub, dst_ub, xu, s_ub_out, deg_ub_out),
            (src_ib, dst_ib, xi, s_ib_out, deg_ib_out)):
        for p in range(NPASS):
            base = (p * NC + c) * R

            for z in range(ROWS_PER_TILE // 16):
                pltpu.sync_copy(
                    zbuf, accum.at[pl.ds(s * ROWS_PER_TILE + z * 16, 16)])
                pltpu.sync_copy(
                    dz, dega.at[pl.ds(s * ROWS_PER_TILE + z * 16, 16)])

            @pl.when(s == 0)
            def _zero_dump():
                pltpu.sync_copy(zbuf, accum.at[pl.ds(R, 16)])
                pltpu.sync_copy(dz, dega.at[pl.ds(R, 16)])

            plsc.subcore_barrier()

            def segment(g, carry):
                e0 = s * EPT + g * SEG
                pltpu.sync_copy(srch.at[pl.ds(e0, SEG)], seg_src)
                pltpu.sync_copy(dsth.at[pl.ds(e0, SEG)], seg_dst)

                def scan_blk(i, cnt):
                    sl = pl.ds(i * LANES, LANES)
                    d16 = seg_dst[sl]
                    s16 = seg_src[sl]
                    off = d16 - base
                    m = (off >= 0) & (off < R)
                    csum = plsc.cumsum(m.astype(jnp.int32))
                    pos = cnt - 1 + csum
                    plsc.store_scatter(gidx, [pos], s16, mask=m)
                    plsc.store_scatter(sidx, [pos], off, mask=m)
                    return cnt + jnp.sum(m.astype(jnp.int32))

                cnt = lax.fori_loop(0, SBLK, scan_blk, jnp.int32(0))

                nch = lax.shift_right_logical(cnt + (CH - 1), 7)

                def fill_blk(j, carry2):
                    pos = j * LANES + iota
                    m = pos >= cnt
                    plsc.store_scatter(gidx, [pos], z16i, mask=m)
                    plsc.store_scatter(sidx, [pos], dump16, mask=m)
                    return carry2

                lax.fori_loop(lax.shift_right_logical(cnt, 4),
                              nch * (CH // LANES), fill_blk, jnp.int32(0))

                nwv = lax.shift_right_logical(nch + (K - 1), 2)

                def wave(w, carry3):
                    c0 = w * K
                    for k in range(K):
                        @pl.when(c0 + k < nch)
                        def _fire(k=k):
                            gsl = gidx.at[pl.ds((c0 + k) * CH, CH)]
                            pltpu.async_copy(table.at[gsl], rows4.at[k], sem)
                    for k in range(K):
                        @pl.when(c0 + k < nch)
                        def _drain(k=k):
                            gsl = gidx.at[pl.ds((c0 + k) * CH, CH)]
                            pltpu.make_async_copy(
                                table.at[gsl], rows4.at[k], sem).wait()
                    for k in range(K):
                        @pl.when(c0 + k < nch)
                        def _scat(k=k):
                            ssl = sidx.at[pl.ds((c0 + k) * CH, CH)]
                            pltpu.sync_copy(rows4.at[k], accum.at[ssl],
                                            add=True)
                            pltpu.sync_copy(ones_v, dega.at[ssl], add=True)
                    return carry3

                lax.fori_loop(0, nwv, wave, jnp.int32(0))
                return carry

            lax.fori_loop(0, NSEG, segment, jnp.int32(0))
            plsc.subcore_barrier()

            row0 = s * ROWS_PER_TILE
            g0 = base + row0
            pltpu.sync_copy(accum.at[pl.ds(row0, ROWS_PER_TILE)],
                            s_out.at[pl.ds(g0, ROWS_PER_TILE)])
            pltpu.sync_copy(dega.at[pl.ds(row0, ROWS_PER_TILE)], deg_stage)
            pltpu.sync_copy(deg_stage, d_out.at[pl.ds(g0, ROWS_PER_TILE)])
            plsc.subcore_barrier()


def _sc_scatter(xu, xi, src_ub, dst_ub, src_ib, dst_ib):
    fn = pl.kernel(
        _sc_body,
        out_type=[
            jax.ShapeDtypeStruct((N_PAD, D), jnp.float32),
            jax.ShapeDtypeStruct((N_PAD,), jnp.float32),
            jax.ShapeDtypeStruct((N_PAD, D), jnp.float32),
            jax.ShapeDtypeStruct((N_PAD,), jnp.float32),
        ],
        mesh=plsc.VectorSubcoreMesh(core_axis_name="c", subcore_axis_name="s"),
        compiler_params=pltpu.CompilerParams(needs_layout_passes=False),
        scratch_types=[
            pltpu.VMEM((SEG,), jnp.int32),
            pltpu.VMEM((SEG,), jnp.int32),
            pltpu.VMEM((SEG + CH,), jnp.int32),
            pltpu.VMEM((SEG + CH,), jnp.int32),
            pltpu.VMEM((K, CH, D), jnp.float32),
            pltpu.VMEM((16, D), jnp.float32),
            pltpu.VMEM((CH,), jnp.float32),
            pltpu.VMEM((16,), jnp.float32),
            pltpu.VMEM((ROWS_PER_TILE,), jnp.float32),
            pltpu.VMEM_SHARED((R + 16, D), jnp.float32),
            pltpu.VMEM_SHARED((R + 16,), jnp.float32),
            pltpu.SemaphoreType.DMA,
        ],
    )
    return fn(xu, xi, src_ub, dst_ub, src_ib, dst_ib)


def _final_body(su, si, du, di, xu, xi, wot, bo, al, out_u, out_i):
    a_l_ub = al[0]
    a_r_ub = al[1]
    a_l_ib = al[2]
    a_r_ib = al[3]
    hi = a_l_ub * su[...] + (a_r_ub * du[...]) * xi[...]
    hi = jnp.where(hi > 0, hi, jnp.exp(hi) - 1.0)
    out_i[...] = hi
    hu = a_l_ib * si[...] + (a_r_ib * di[...]) * xu[...]
    hu = jnp.where(hu > 0, hu, jnp.exp(hu) - 1.0)
    out_u[...] = jnp.dot(hu, wot[...],
                         preferred_element_type=jnp.float32) + bo[...]


def _finalize(s_ub, s_ib, deg_ub, deg_ib, xu, xi, wot, bo, al):
    return pl.pallas_call(
        _final_body,
        grid=(NRB,),
        in_specs=[
            pl.BlockSpec((ROWBLK, D), lambda i: (i, 0)),
            pl.BlockSpec((ROWBLK, D), lambda i: (i, 0)),
            pl.BlockSpec((ROWBLK, 1), lambda i: (i, 0)),
            pl.BlockSpec((ROWBLK, 1), lambda i: (i, 0)),
            pl.BlockSpec((ROWBLK, D), lambda i: (i, 0)),
            pl.BlockSpec((ROWBLK, D), lambda i: (i, 0)),
            pl.BlockSpec((D, D), lambda i: (0, 0)),
            pl.BlockSpec((1, D), lambda i: (0, 0)),
            pl.BlockSpec(memory_space=pltpu.SMEM),
        ],
        out_specs=[
            pl.BlockSpec((ROWBLK, D), lambda i: (i, 0)),
            pl.BlockSpec((ROWBLK, D), lambda i: (i, 0)),
        ],
        out_shape=[jax.ShapeDtypeStruct((N, D), jnp.float32)] * 2,
    )(s_ub, s_ib, deg_ub, deg_ib, xu, xi, wot, bo, al)


def kernel(x_user, x_item, edge_index_ub, edge_index_ib, W_proj, b_proj,
           alpha_l_ub, alpha_r_ub, alpha_l_ib, alpha_r_ib, W_out, b_out):
    xu, xi = _project(x_user, x_item, W_proj.T, b_proj.reshape(1, D))

    pad_src = jnp.zeros((E_PAD - E,), jnp.int32)
    pad_dst = jnp.full((E_PAD - E,), N_PAD - 1, jnp.int32)
    src_ub = jnp.concatenate([edge_index_ub[0], pad_src])
    dst_ub = jnp.concatenate([edge_index_ub[1], pad_dst])
    src_ib = jnp.concatenate([edge_index_ib[0], pad_src])
    dst_ib = jnp.concatenate([edge_index_ib[1], pad_dst])

    s_ub, deg_ub, s_ib, deg_ib = _sc_scatter(
        xu, xi, src_ub, dst_ub, src_ib, dst_ib)

    al = jnp.stack([alpha_l_ub[0], alpha_r_ub[0],
                    alpha_l_ib[0], alpha_r_ib[0]])
    out_u, out_i = _finalize(
        s_ub, s_ib, deg_ub.reshape(N_PAD, 1), deg_ib.reshape(N_PAD, 1),
        xu, xi, W_out.T, b_out.reshape(1, D), al)
    return (out_u, out_i)


# speedup vs baseline: 5.3519x; 5.3519x over previous
import jax
import jax.numpy as jnp
from jax import lax
from jax.experimental import pallas as pl
from jax.experimental.pallas import tpu as pltpu
from jax.experimental.pallas import tpu_sc as plsc

N = 50000
D = 128
E = 400000

NC = 2
NS = 16
LANES = 16

ROWS_PER_TILE = 1568
R = NS * ROWS_PER_TILE
NPASS = 1
N_PAD = NC * R * NPASS
SEG = 1792
NSEG = 14
EPT = SEG * NSEG
E_PAD = EPT * NS
SBLK = SEG // LANES
CH = 128
DUMP = R
ROWBLK = 2000
NRB = N // ROWBLK


def _proj_body(xu_in, xi_in, wt, b, xu_out, xi_out, xu_bf_out, xi_bf_out):
    bb = b[...]
    xu = jnp.dot(xu_in[...], wt[...], preferred_element_type=jnp.float32) + bb
    xi = jnp.dot(xi_in[...], wt[...], preferred_element_type=jnp.float32) + bb
    xu_out[...] = xu
    xi_out[...] = xi
    xu_bf_out[...] = xu.astype(jnp.bfloat16)
    xi_bf_out[...] = xi.astype(jnp.bfloat16)


def _project(x_user, x_item, wt, b):
    return pl.pallas_call(
        _proj_body,
        grid=(NRB,),
        in_specs=[
            pl.BlockSpec((ROWBLK, D), lambda i: (i, 0)),
            pl.BlockSpec((ROWBLK, D), lambda i: (i, 0)),
            pl.BlockSpec((D, D), lambda i: (0, 0)),
            pl.BlockSpec((1, D), lambda i: (0, 0)),
        ],
        out_specs=[
            pl.BlockSpec((ROWBLK, D), lambda i: (i, 0)),
            pl.BlockSpec((ROWBLK, D), lambda i: (i, 0)),
            pl.BlockSpec((ROWBLK, D), lambda i: (i, 0)),
            pl.BlockSpec((ROWBLK, D), lambda i: (i, 0)),
        ],
        out_shape=[jax.ShapeDtypeStruct((N, D), jnp.float32)] * 2
        + [jax.ShapeDtypeStruct((N, D), jnp.bfloat16)] * 2,
    )(x_user, x_item, wt, b)


def _sc_body(xu, xi, src_ub, dst_ub, src_ib, dst_ib,
             s_ub_out, deg_ub_out, s_ib_out, deg_ib_out,
             seg_src, seg_dst, gidx, sidx, rows, zbuf, ones_v,
             dz, deg_stage, accum, dega, sem):
    c = lax.axis_index("c")
    s = lax.axis_index("s")
    iota = lax.iota(jnp.int32, LANES)
    z16f = jnp.zeros((LANES,), jnp.float32)
    z32bf = jnp.zeros((2 * LANES,), jnp.bfloat16)
    dumpg16 = jnp.full((LANES,), s * ROWS_PER_TILE, jnp.int32)
    dump16 = jnp.full((LANES,), DUMP, jnp.int32)

    for r_ in range(16):
        for cb in range(D // (2 * LANES)):
            zbuf[r_, pl.ds(cb * 2 * LANES, 2 * LANES)] = z32bf
    for cb in range(CH // LANES):
        ones_v[pl.ds(cb * LANES, LANES)] = jnp.ones((LANES,), jnp.float32)
    dz[pl.ds(0, LANES)] = z16f

    for (srch, dsth, table, s_out, d_out) in (
            (src_ub, dst_ub, xu, s_ub_out, deg_ub_out),
            (src_ib, dst_ib, xi, s_ib_out, deg_ib_out)):
        for p in range(NPASS):
            base = (p * NC + c) * R

            for z in range(ROWS_PER_TILE // 16):
                pltpu.sync_copy(
                    zbuf, accum.at[pl.ds(s * ROWS_PER_TILE + z * 16, 16)])
                pltpu.sync_copy(
                    dz, dega.at[pl.ds(s * ROWS_PER_TILE + z * 16, 16)])

            @pl.when(s == 0)
            def _zero_dump():
                pltpu.sync_copy(zbuf, accum.at[pl.ds(R, 16)])
                pltpu.sync_copy(dz, dega.at[pl.ds(R, 16)])

            plsc.subcore_barrier()

            def segment(g, carry):
                e0 = s * EPT + g * SEG
                pltpu.sync_copy(srch.at[pl.ds(e0, SEG)], seg_src)
                pltpu.sync_copy(dsth.at[pl.ds(e0, SEG)], seg_dst)

                def scan_blk(i, cnt):
                    sl = pl.ds(i * LANES, LANES)
                    d16 = seg_dst[sl]
                    s16 = seg_src[sl]
                    off = d16 - base
                    m = (off >= 0) & (off < R)
                    csum = plsc.cumsum(m.astype(jnp.int32))
                    pos = cnt - 1 + csum
                    plsc.store_scatter(gidx, [pos], s16, mask=m)
                    plsc.store_scatter(sidx, [pos], off, mask=m)
                    return cnt + jnp.sum(m.astype(jnp.int32))

                cnt = lax.fori_loop(0, SBLK, scan_blk, jnp.int32(0))

                nch = lax.shift_right_logical(cnt + (CH - 1), 7)

                def fill_blk(j, carry2):
                    pos = j * LANES + iota
                    m = pos >= cnt
                    plsc.store_scatter(gidx, [pos], dumpg16, mask=m)
                    plsc.store_scatter(sidx, [pos], dump16, mask=m)
                    return carry2

                lax.fori_loop(lax.shift_right_logical(cnt, 4),
                              nch * (CH // LANES), fill_blk, jnp.int32(0))

                def chunk(cc, carry3):
                    gsl = gidx.at[pl.ds(cc * CH, CH)]
                    ssl = sidx.at[pl.ds(cc * CH, CH)]
                    pltpu.async_copy(table.at[gsl], rows, sem).wait()
                    pltpu.sync_copy(rows, accum.at[ssl], add=True)
                    pltpu.sync_copy(ones_v, dega.at[ssl], add=True)
                    return carry3

                lax.fori_loop(0, nch, chunk, jnp.int32(0))
                return carry

            lax.fori_loop(0, NSEG, segment, jnp.int32(0))
            plsc.subcore_barrier()

            row0 = s * ROWS_PER_TILE
            g0 = base + row0
            pltpu.sync_copy(accum.at[pl.ds(row0, ROWS_PER_TILE)],
                            s_out.at[pl.ds(g0, ROWS_PER_TILE)])
            pltpu.sync_copy(dega.at[pl.ds(row0, ROWS_PER_TILE)], deg_stage)
            pltpu.sync_copy(deg_stage, d_out.at[pl.ds(g0, ROWS_PER_TILE)])
            plsc.subcore_barrier()


def _sc_scatter(xu, xi, src_ub, dst_ub, src_ib, dst_ib):
    fn = pl.kernel(
        _sc_body,
        out_type=[
            jax.ShapeDtypeStruct((N_PAD, D), jnp.bfloat16),
            jax.ShapeDtypeStruct((N_PAD,), jnp.float32),
            jax.ShapeDtypeStruct((N_PAD, D), jnp.bfloat16),
            jax.ShapeDtypeStruct((N_PAD,), jnp.float32),
        ],
        mesh=plsc.VectorSubcoreMesh(core_axis_name="c", subcore_axis_name="s"),
        compiler_params=pltpu.CompilerParams(needs_layout_passes=False, use_tc_tiling_on_sc=False),
        scratch_types=[
            pltpu.VMEM((SEG,), jnp.int32),
            pltpu.VMEM((SEG,), jnp.int32),
            pltpu.VMEM((SEG + CH,), jnp.int32),
            pltpu.VMEM((SEG + CH,), jnp.int32),
            pltpu.VMEM((CH, D), jnp.bfloat16),
            pltpu.VMEM((16, D), jnp.bfloat16),
            pltpu.VMEM((CH,), jnp.float32),
            pltpu.VMEM((16,), jnp.float32),
            pltpu.VMEM((ROWS_PER_TILE,), jnp.float32),
            pltpu.VMEM_SHARED((R + 16, D), jnp.bfloat16),
            pltpu.VMEM_SHARED((R + 16,), jnp.float32),
            pltpu.SemaphoreType.DMA,
        ],
    )
    return fn(xu, xi, src_ub, dst_ub, src_ib, dst_ib)


def _final_body(su, si, du, di, xu, xi, wot, bo, al, out_u, out_i):
    a_l_ub = al[0]
    a_r_ub = al[1]
    a_l_ib = al[2]
    a_r_ib = al[3]
    hi = a_l_ub * su[...].astype(jnp.float32) + (a_r_ub * du[...]) * xi[...]
    hi = jnp.where(hi > 0, hi, jnp.exp(hi) - 1.0)
    out_i[...] = hi
    hu = a_l_ib * si[...].astype(jnp.float32) + (a_r_ib * di[...]) * xu[...]
    hu = jnp.where(hu > 0, hu, jnp.exp(hu) - 1.0)
    out_u[...] = jnp.dot(hu, wot[...],
                         preferred_element_type=jnp.float32) + bo[...]


def _finalize(s_ub, s_ib, deg_ub, deg_ib, xu, xi, wot, bo, al):
    return pl.pallas_call(
        _final_body,
        grid=(NRB,),
        in_specs=[
            pl.BlockSpec((ROWBLK, D), lambda i: (i, 0)),
            pl.BlockSpec((ROWBLK, D), lambda i: (i, 0)),
            pl.BlockSpec((ROWBLK, 1), lambda i: (i, 0)),
            pl.BlockSpec((ROWBLK, 1), lambda i: (i, 0)),
            pl.BlockSpec((ROWBLK, D), lambda i: (i, 0)),
            pl.BlockSpec((ROWBLK, D), lambda i: (i, 0)),
            pl.BlockSpec((D, D), lambda i: (0, 0)),
            pl.BlockSpec((1, D), lambda i: (0, 0)),
            pl.BlockSpec(memory_space=pltpu.SMEM),
        ],
        out_specs=[
            pl.BlockSpec((ROWBLK, D), lambda i: (i, 0)),
            pl.BlockSpec((ROWBLK, D), lambda i: (i, 0)),
        ],
        out_shape=[jax.ShapeDtypeStruct((N, D), jnp.float32)] * 2,
    )(s_ub, s_ib, deg_ub, deg_ib, xu, xi, wot, bo, al)


def kernel(x_user, x_item, edge_index_ub, edge_index_ib, W_proj, b_proj,
           alpha_l_ub, alpha_r_ub, alpha_l_ib, alpha_r_ib, W_out, b_out):
    xu, xi, xu_bf, xi_bf = _project(x_user, x_item, W_proj.T,
                                    b_proj.reshape(1, D))

    pad_src = jnp.zeros((E_PAD - E,), jnp.int32)
    pad_dst = jnp.full((E_PAD - E,), N_PAD - 1, jnp.int32)
    src_ub = jnp.concatenate([edge_index_ub[0], pad_src])
    dst_ub = jnp.concatenate([edge_index_ub[1], pad_dst])
    src_ib = jnp.concatenate([edge_index_ib[0], pad_src])
    dst_ib = jnp.concatenate([edge_index_ib[1], pad_dst])

    s_ub, deg_ub, s_ib, deg_ib = _sc_scatter(
        xu_bf, xi_bf, src_ub, dst_ub, src_ib, dst_ib)

    al = jnp.stack([alpha_l_ub[0], alpha_r_ub[0],
                    alpha_l_ib[0], alpha_r_ib[0]])
    out_u, out_i = _finalize(
        s_ub, s_ib, deg_ub.reshape(N_PAD, 1), deg_ib.reshape(N_PAD, 1),
        xu, xi, W_out.T, b_out.reshape(1, D), al)
    return (out_u, out_i)


# speedup vs baseline: 6.2873x; 1.1748x over previous
import jax
import jax.numpy as jnp
from jax import lax
from jax.experimental import pallas as pl
from jax.experimental.pallas import tpu as pltpu
from jax.experimental.pallas import tpu_sc as plsc

N = 50000
D = 128
E = 400000

NC = 2
NS = 16
LANES = 16

ROWS_PER_TILE = 1568
R = NS * ROWS_PER_TILE
NPASS = 1
N_PAD = NC * R * NPASS
SEG = 1792
NSEG = 14
EPT = SEG * NSEG
E_PAD = EPT * NS
SBLK = SEG // LANES
CH = 128
DUMP = R
ROWBLK = 2000
NRB = N // ROWBLK


_DNT = (((1,), (1,)), ((), ()))


def _proj_body(xu_in, xi_in, wt, b, xu_out, xi_out, xu_bf_out, xi_bf_out):
    bb = b[...]
    xu = lax.dot_general(xu_in[...], wt[...], _DNT,
                         preferred_element_type=jnp.float32) + bb
    xi = lax.dot_general(xi_in[...], wt[...], _DNT,
                         preferred_element_type=jnp.float32) + bb
    xu_out[...] = xu
    xi_out[...] = xi
    xu_bf_out[...] = xu.astype(jnp.bfloat16)
    xi_bf_out[...] = xi.astype(jnp.bfloat16)


def _project(x_user, x_item, wt, b):
    return pl.pallas_call(
        _proj_body,
        grid=(NRB,),
        in_specs=[
            pl.BlockSpec((ROWBLK, D), lambda i: (i, 0)),
            pl.BlockSpec((ROWBLK, D), lambda i: (i, 0)),
            pl.BlockSpec((D, D), lambda i: (0, 0)),
            pl.BlockSpec((1, D), lambda i: (0, 0)),
        ],
        out_specs=[
            pl.BlockSpec((ROWBLK, D), lambda i: (i, 0)),
            pl.BlockSpec((ROWBLK, D), lambda i: (i, 0)),
            pl.BlockSpec((ROWBLK, D), lambda i: (i, 0)),
            pl.BlockSpec((ROWBLK, D), lambda i: (i, 0)),
        ],
        out_shape=[jax.ShapeDtypeStruct((N, D), jnp.float32)] * 2
        + [jax.ShapeDtypeStruct((N, D), jnp.bfloat16)] * 2,
    )(x_user, x_item, wt, b)


def _sc_body(xu, xi, edges_ub, edges_ib,
             s_ub_out, deg_ub_out, s_ib_out, deg_ib_out,
             seg_src, seg_dst, gidx, sidx, rows0, rows1, zbuf, ones_v,
             dz, deg_stage, accum, dega, sem0, sem1):
    c = lax.axis_index("c")
    s = lax.axis_index("s")
    iota = lax.iota(jnp.int32, LANES)
    z16f = jnp.zeros((LANES,), jnp.float32)
    z32bf = jnp.zeros((2 * LANES,), jnp.bfloat16)
    dumpg16 = jnp.full((LANES,), s * ROWS_PER_TILE, jnp.int32)
    dump16 = jnp.full((LANES,), DUMP, jnp.int32)

    for r_ in range(16):
        for cb in range(D // (2 * LANES)):
            zbuf[r_, pl.ds(cb * 2 * LANES, 2 * LANES)] = z32bf
    for cb in range(CH // LANES):
        ones_v[pl.ds(cb * LANES, LANES)] = jnp.ones((LANES,), jnp.float32)
    dz[pl.ds(0, LANES)] = z16f

    for (edges, table, s_out, d_out) in (
            (edges_ub, xu, s_ub_out, deg_ub_out),
            (edges_ib, xi, s_ib_out, deg_ib_out)):
        for p in range(NPASS):
            base = (p * NC + c) * R

            for z in range(ROWS_PER_TILE // 16):
                pltpu.sync_copy(
                    zbuf, accum.at[pl.ds(s * ROWS_PER_TILE + z * 16, 16)])
                pltpu.sync_copy(
                    dz, dega.at[pl.ds(s * ROWS_PER_TILE + z * 16, 16)])

            @pl.when(s == 0)
            def _zero_dump():
                pltpu.sync_copy(zbuf, accum.at[pl.ds(R, 16)])
                pltpu.sync_copy(dz, dega.at[pl.ds(R, 16)])

            plsc.subcore_barrier()

            def segment(g, carry):
                e0 = s * EPT + g * SEG
                pltpu.sync_copy(edges.at[0, pl.ds(e0, SEG)], seg_src)
                pltpu.sync_copy(edges.at[1, pl.ds(e0, SEG)], seg_dst)

                def scan_blk(i, cnt):
                    sl = pl.ds(i * LANES, LANES)
                    d16 = seg_dst[sl]
                    s16 = seg_src[sl]
                    off = d16 - base
                    m = (off >= 0) & (off < R)
                    csum = plsc.cumsum(m.astype(jnp.int32))
                    pos = cnt - 1 + csum
                    plsc.store_scatter(gidx, [pos], s16, mask=m)
                    plsc.store_scatter(sidx, [pos], off, mask=m)
                    return cnt + jnp.sum(m.astype(jnp.int32))

                cnt = lax.fori_loop(0, SBLK, scan_blk, jnp.int32(0))

                nch = lax.shift_right_logical(cnt + (CH - 1), 7)

                def fill_blk(j, carry2):
                    pos = j * LANES + iota
                    m = pos >= cnt
                    plsc.store_scatter(gidx, [pos], dumpg16, mask=m)
                    plsc.store_scatter(sidx, [pos], dump16, mask=m)
                    return carry2

                lax.fori_loop(lax.shift_right_logical(cnt, 4),
                              nch * (CH // LANES), fill_blk, jnp.int32(0))

                def _fire(cc, buf, sem):
                    pltpu.async_copy(
                        table.at[gidx.at[pl.ds(cc * CH, CH)]], buf, sem)

                def _drain(cc, buf, sem):
                    pltpu.make_async_copy(
                        table.at[gidx.at[pl.ds(cc * CH, CH)]], buf, sem).wait()

                def _scat(cc, buf):
                    ssl = sidx.at[pl.ds(cc * CH, CH)]
                    pltpu.sync_copy(buf, accum.at[ssl], add=True)
                    pltpu.sync_copy(ones_v, dega.at[ssl], add=True)

                @pl.when(nch > 0)
                def _prologue():
                    _fire(jnp.int32(0), rows0, sem0)

                def pair(pp, carry3):
                    c0 = pp * 2
                    _drain(c0, rows0, sem0)

                    @pl.when(c0 + 1 < nch)
                    def _f1():
                        _fire(c0 + 1, rows1, sem1)

                    _scat(c0, rows0)

                    @pl.when(c0 + 2 < nch)
                    def _f2():
                        _fire(c0 + 2, rows0, sem0)

                    @pl.when(c0 + 1 < nch)
                    def _s1():
                        _drain(c0 + 1, rows1, sem1)
                        _scat(c0 + 1, rows1)

                    return carry3

                lax.fori_loop(0, lax.shift_right_logical(nch + 1, 1), pair,
                              jnp.int32(0))
                return carry

            lax.fori_loop(0, NSEG, segment, jnp.int32(0))
            plsc.subcore_barrier()

            row0 = s * ROWS_PER_TILE
            g0 = base + row0
            pltpu.sync_copy(accum.at[pl.ds(row0, ROWS_PER_TILE)],
                            s_out.at[pl.ds(g0, ROWS_PER_TILE)])
            pltpu.sync_copy(dega.at[pl.ds(row0, ROWS_PER_TILE)], deg_stage)
            pltpu.sync_copy(deg_stage, d_out.at[pl.ds(g0, ROWS_PER_TILE)])
            plsc.subcore_barrier()


def _sc_scatter(xu, xi, edges_ub, edges_ib):
    fn = pl.kernel(
        _sc_body,
        out_type=[
            jax.ShapeDtypeStruct((N_PAD, D), jnp.bfloat16),
            jax.ShapeDtypeStruct((N_PAD,), jnp.float32),
            jax.ShapeDtypeStruct((N_PAD, D), jnp.bfloat16),
            jax.ShapeDtypeStruct((N_PAD,), jnp.float32),
        ],
        mesh=plsc.VectorSubcoreMesh(core_axis_name="c", subcore_axis_name="s"),
        compiler_params=pltpu.CompilerParams(needs_layout_passes=False, use_tc_tiling_on_sc=False),
        scratch_types=[
            pltpu.VMEM((SEG,), jnp.int32),
            pltpu.VMEM((SEG,), jnp.int32),
            pltpu.VMEM((SEG + CH,), jnp.int32),
            pltpu.VMEM((SEG + CH,), jnp.int32),
            pltpu.VMEM((CH, D), jnp.bfloat16),
            pltpu.VMEM((CH, D), jnp.bfloat16),
            pltpu.VMEM((16, D), jnp.bfloat16),
            pltpu.VMEM((CH,), jnp.float32),
            pltpu.VMEM((16,), jnp.float32),
            pltpu.VMEM((ROWS_PER_TILE,), jnp.float32),
            pltpu.VMEM_SHARED((R + 16, D), jnp.bfloat16),
            pltpu.VMEM_SHARED((R + 16,), jnp.float32),
            pltpu.SemaphoreType.DMA,
            pltpu.SemaphoreType.DMA,
        ],
    )
    return fn(xu, xi, edges_ub, edges_ib)


def _final_body(su, si, du, di, xu, xi, wo, bo, alu, aru, ali, ari,
                out_u, out_i):
    a_l_ub = alu[0]
    a_r_ub = aru[0]
    a_l_ib = ali[0]
    a_r_ib = ari[0]
    hi = a_l_ub * su[...].astype(jnp.float32) + (a_r_ub * du[...]) * xi[...]
    hi = jnp.where(hi > 0, hi, jnp.exp(hi) - 1.0)
    out_i[...] = hi
    hu = a_l_ib * si[...].astype(jnp.float32) + (a_r_ib * di[...]) * xu[...]
    hu = jnp.where(hu > 0, hu, jnp.exp(hu) - 1.0)
    out_u[...] = lax.dot_general(hu, wo[...], _DNT,
                                 preferred_element_type=jnp.float32) + bo[...]


def _finalize(s_ub, s_ib, deg_ub, deg_ib, xu, xi, wo, bo,
              alu, aru, ali, ari):
    return pl.pallas_call(
        _final_body,
        grid=(NRB,),
        in_specs=[
            pl.BlockSpec((ROWBLK, D), lambda i: (i, 0)),
            pl.BlockSpec((ROWBLK, D), lambda i: (i, 0)),
            pl.BlockSpec((ROWBLK, 1), lambda i: (i, 0)),
            pl.BlockSpec((ROWBLK, 1), lambda i: (i, 0)),
            pl.BlockSpec((ROWBLK, D), lambda i: (i, 0)),
            pl.BlockSpec((ROWBLK, D), lambda i: (i, 0)),
            pl.BlockSpec((D, D), lambda i: (0, 0)),
            pl.BlockSpec((1, D), lambda i: (0, 0)),
            pl.BlockSpec(memory_space=pltpu.SMEM),
            pl.BlockSpec(memory_space=pltpu.SMEM),
            pl.BlockSpec(memory_space=pltpu.SMEM),
            pl.BlockSpec(memory_space=pltpu.SMEM),
        ],
        out_specs=[
            pl.BlockSpec((ROWBLK, D), lambda i: (i, 0)),
            pl.BlockSpec((ROWBLK, D), lambda i: (i, 0)),
        ],
        out_shape=[jax.ShapeDtypeStruct((N, D), jnp.float32)] * 2,
    )(s_ub, s_ib, deg_ub, deg_ib, xu, xi, wo, bo, alu, aru, ali, ari)


def kernel(x_user, x_item, edge_index_ub, edge_index_ib, W_proj, b_proj,
           alpha_l_ub, alpha_r_ub, alpha_l_ib, alpha_r_ib, W_out, b_out):
    xu, xi, xu_bf, xi_bf = _project(x_user, x_item, W_proj,
                                    b_proj.reshape(1, D))

    pad = jnp.stack([jnp.zeros((E_PAD - E,), jnp.int32),
                     jnp.full((E_PAD - E,), N_PAD - 1, jnp.int32)])
    edges_ub = jnp.concatenate([edge_index_ub, pad], axis=1)
    edges_ib = jnp.concatenate([edge_index_ib, pad], axis=1)

    s_ub, deg_ub, s_ib, deg_ib = _sc_scatter(xu_bf, xi_bf, edges_ub, edges_ib)

    out_u, out_i = _finalize(
        s_ub, s_ib, deg_ub.reshape(N_PAD, 1), deg_ib.reshape(N_PAD, 1),
        xu, xi, W_out, b_out.reshape(1, D),
        alpha_l_ub, alpha_r_ub, alpha_l_ib, alpha_r_ib)
    return (out_u, out_i)


# speedup vs baseline: 6.7254x; 1.0697x over previous
import jax
import jax.numpy as jnp
from jax import lax
from jax.experimental import pallas as pl
from jax.experimental.pallas import tpu as pltpu
from jax.experimental.pallas import tpu_sc as plsc

N = 50000
D = 128
E = 400000

NC = 2
NS = 16
LANES = 16

ROWS_PER_TILE = 1568
R = NS * ROWS_PER_TILE
NPASS = 1
N_PAD = NC * R * NPASS
SEG = 1792
NSEG = 14
EPT = SEG * NSEG
E_PAD = EPT * NS
SBLK = SEG // LANES
CH = 128
DUMP = R
ROWBLK = 2000
NRB = N // ROWBLK


_DNT = (((1,), (1,)), ((), ()))


def _proj_body(xu_in, xi_in, wt, b, xu_out, xi_out, xu_bf_out, xi_bf_out):
    bb = b[...]
    xu = lax.dot_general(xu_in[...], wt[...], _DNT,
                         preferred_element_type=jnp.float32) + bb
    xi = lax.dot_general(xi_in[...], wt[...], _DNT,
                         preferred_element_type=jnp.float32) + bb
    xu_out[...] = xu
    xi_out[...] = xi
    xu_bf_out[...] = xu.astype(jnp.bfloat16)
    xi_bf_out[...] = xi.astype(jnp.bfloat16)


def _project(x_user, x_item, wt, b):
    return pl.pallas_call(
        _proj_body,
        grid=(NRB,),
        in_specs=[
            pl.BlockSpec((ROWBLK, D), lambda i: (i, 0)),
            pl.BlockSpec((ROWBLK, D), lambda i: (i, 0)),
            pl.BlockSpec((D, D), lambda i: (0, 0)),
            pl.BlockSpec((1, D), lambda i: (0, 0)),
        ],
        out_specs=[
            pl.BlockSpec((ROWBLK, D), lambda i: (i, 0)),
            pl.BlockSpec((ROWBLK, D), lambda i: (i, 0)),
            pl.BlockSpec((ROWBLK, D), lambda i: (i, 0)),
            pl.BlockSpec((ROWBLK, D), lambda i: (i, 0)),
        ],
        out_shape=[jax.ShapeDtypeStruct((N, D), jnp.float32)] * 2
        + [jax.ShapeDtypeStruct((N, D), jnp.bfloat16)] * 2,
    )(x_user, x_item, wt, b)


def _sc_body(xu, xi, edges_ub, edges_ib,
             s_ub_out, deg_ub_out, s_ib_out, deg_ib_out,
             seg_src0, seg_dst0, seg_src1, seg_dst1, gidx, sidx, rows0,
             rows1, ones_v, dz, deg_stage, accum, dega,
             sem0, sem1, semd, seme0, seme1):
    c = lax.axis_index("c")
    s = lax.axis_index("s")
    iota = lax.iota(jnp.int32, LANES)
    z16f = jnp.zeros((LANES,), jnp.float32)
    z32bf = jnp.zeros((2 * LANES,), jnp.bfloat16)
    dumpg16 = jnp.full((LANES,), s * ROWS_PER_TILE, jnp.int32)
    dump16 = jnp.full((LANES,), DUMP, jnp.int32)

    for cb in range(CH // LANES):
        ones_v[pl.ds(cb * LANES, LANES)] = jnp.ones((LANES,), jnp.float32)
    for cb in range(CH // LANES):
        dz[pl.ds(cb * LANES, LANES)] = z16f

    for (edges, table, s_out, d_out) in (
            (edges_ub, xu, s_ub_out, deg_ub_out),
            (edges_ib, xi, s_ib_out, deg_ib_out)):
        for p in range(NPASS):
            base = (p * NC + c) * R
            row00 = s * ROWS_PER_TILE

            for r_ in range(CH):
                for cb in range(D // (2 * LANES)):
                    rows0[r_, pl.ds(cb * 2 * LANES, 2 * LANES)] = z32bf
            for z in range(ROWS_PER_TILE // CH):
                pltpu.sync_copy(rows0, accum.at[pl.ds(row00 + z * CH, CH)])
            pltpu.sync_copy(
                rows0.at[pl.ds(0, ROWS_PER_TILE % CH)],
                accum.at[pl.ds(row00 + (ROWS_PER_TILE // CH) * CH,
                               ROWS_PER_TILE % CH)])
            for z in range(ROWS_PER_TILE // CH):
                pltpu.sync_copy(dz, dega.at[pl.ds(row00 + z * CH, CH)])
            pltpu.sync_copy(
                dz.at[pl.ds(0, ROWS_PER_TILE % CH)],
                dega.at[pl.ds(row00 + (ROWS_PER_TILE // CH) * CH,
                              ROWS_PER_TILE % CH)])

            @pl.when(s == 0)
            def _zero_dump():
                pltpu.sync_copy(rows0.at[pl.ds(0, 16)],
                                accum.at[pl.ds(R, 16)])
                pltpu.sync_copy(dz.at[pl.ds(0, 16)], dega.at[pl.ds(R, 16)])

            plsc.subcore_barrier()

            def fire_seg(g, ssb, sdb, sme):
                e0 = s * EPT + g * SEG
                pltpu.async_copy(edges.at[0, pl.ds(e0, SEG)], ssb, sme)
                pltpu.async_copy(edges.at[1, pl.ds(e0, SEG)], sdb, sme)

            def drain_seg(g, ssb, sdb, sme):
                e0 = s * EPT + g * SEG
                pltpu.make_async_copy(
                    edges.at[0, pl.ds(e0, SEG)], ssb, sme).wait()
                pltpu.make_async_copy(
                    edges.at[1, pl.ds(e0, SEG)], sdb, sme).wait()

            def segment(g, seg_src, seg_dst):
                def scan_blk(i, cnt):
                    sl = pl.ds(i * LANES, LANES)
                    d16 = seg_dst[sl]
                    s16 = seg_src[sl]
                    off = d16 - base
                    m = (off >= 0) & (off < R)
                    csum = plsc.cumsum(m.astype(jnp.int32))
                    pos = cnt - 1 + csum
                    plsc.store_scatter(gidx, [pos], s16, mask=m)
                    plsc.store_scatter(sidx, [pos], off, mask=m)
                    return cnt + jnp.sum(m.astype(jnp.int32))

                cnt = lax.fori_loop(0, SBLK, scan_blk, jnp.int32(0))

                nch = lax.shift_right_logical(cnt + (CH - 1), 7)

                def fill_blk(j, carry2):
                    pos = j * LANES + iota
                    m = pos >= cnt
                    plsc.store_scatter(gidx, [pos], dumpg16, mask=m)
                    plsc.store_scatter(sidx, [pos], dump16, mask=m)
                    return carry2

                lax.fori_loop(lax.shift_right_logical(cnt, 4),
                              nch * (CH // LANES), fill_blk, jnp.int32(0))

                def _fire(cc, buf, sem):
                    pltpu.async_copy(
                        table.at[gidx.at[pl.ds(cc * CH, CH)]], buf, sem)

                def _drain(cc, buf, sem):
                    pltpu.make_async_copy(
                        table.at[gidx.at[pl.ds(cc * CH, CH)]], buf, sem).wait()

                def _scat(cc, buf):
                    ssl = sidx.at[pl.ds(cc * CH, CH)]
                    pltpu.sync_copy(buf, accum.at[ssl], add=True)
                    pltpu.async_copy(ones_v, dega.at[ssl], semd, add=True)

                @pl.when(nch > 0)
                def _prologue():
                    _fire(jnp.int32(0), rows0, sem0)

                def pair(pp, carry3):
                    c0 = pp * 2
                    _drain(c0, rows0, sem0)

                    @pl.when(c0 + 1 < nch)
                    def _f1():
                        _fire(c0 + 1, rows1, sem1)

                    _scat(c0, rows0)

                    @pl.when(c0 + 2 < nch)
                    def _f2():
                        _fire(c0 + 2, rows0, sem0)

                    @pl.when(c0 + 1 < nch)
                    def _s1():
                        _drain(c0 + 1, rows1, sem1)
                        _scat(c0 + 1, rows1)

                    return carry3

                lax.fori_loop(0, lax.shift_right_logical(nch + 1, 1), pair,
                              jnp.int32(0))

                def ddrain(cc, carry4):
                    pltpu.make_async_copy(
                        ones_v, dega.at[pl.ds(0, CH)], semd).wait()
                    return carry4

                lax.fori_loop(0, nch, ddrain, jnp.int32(0))

            fire_seg(0, seg_src0, seg_dst0, seme0)
            fire_seg(1, seg_src1, seg_dst1, seme1)

            def segpair(q, carry):
                g0 = q * 2
                drain_seg(g0, seg_src0, seg_dst0, seme0)
                segment(g0, seg_src0, seg_dst0)

                @pl.when(g0 + 2 < NSEG)
                def _pf0():
                    fire_seg(g0 + 2, seg_src0, seg_dst0, seme0)

                drain_seg(g0 + 1, seg_src1, seg_dst1, seme1)
                segment(g0 + 1, seg_src1, seg_dst1)

                @pl.when(g0 + 3 < NSEG)
                def _pf1():
                    fire_seg(g0 + 3, seg_src1, seg_dst1, seme1)

                return carry

            lax.fori_loop(0, NSEG // 2, segpair, jnp.int32(0))
            plsc.subcore_barrier()

            row0 = s * ROWS_PER_TILE
            g0 = base + row0
            pltpu.sync_copy(accum.at[pl.ds(row0, ROWS_PER_TILE)],
                            s_out.at[pl.ds(g0, ROWS_PER_TILE)])
            pltpu.sync_copy(dega.at[pl.ds(row0, ROWS_PER_TILE)], deg_stage)
            pltpu.sync_copy(deg_stage, d_out.at[pl.ds(g0, ROWS_PER_TILE)])
            plsc.subcore_barrier()


def _sc_scatter(xu, xi, edges_ub, edges_ib):
    fn = pl.kernel(
        _sc_body,
        out_type=[
            jax.ShapeDtypeStruct((N_PAD, D), jnp.bfloat16),
            jax.ShapeDtypeStruct((N_PAD,), jnp.float32),
            jax.ShapeDtypeStruct((N_PAD, D), jnp.bfloat16),
            jax.ShapeDtypeStruct((N_PAD,), jnp.float32),
        ],
        mesh=plsc.VectorSubcoreMesh(core_axis_name="c", subcore_axis_name="s"),
        compiler_params=pltpu.CompilerParams(needs_layout_passes=False, use_tc_tiling_on_sc=False),
        scratch_types=[
            pltpu.VMEM((SEG,), jnp.int32),
            pltpu.VMEM((SEG,), jnp.int32),
            pltpu.VMEM((SEG,), jnp.int32),
            pltpu.VMEM((SEG,), jnp.int32),
            pltpu.VMEM((SEG,), jnp.int32),
            pltpu.VMEM((SEG,), jnp.int32),
            pltpu.VMEM((CH, D), jnp.bfloat16),
            pltpu.VMEM((CH, D), jnp.bfloat16),
            pltpu.VMEM((CH,), jnp.float32),
            pltpu.VMEM((CH,), jnp.float32),
            pltpu.VMEM((ROWS_PER_TILE,), jnp.float32),
            pltpu.VMEM_SHARED((R + 16, D), jnp.bfloat16),
            pltpu.VMEM_SHARED((R + 16,), jnp.float32),
            pltpu.SemaphoreType.DMA,
            pltpu.SemaphoreType.DMA,
            pltpu.SemaphoreType.DMA,
            pltpu.SemaphoreType.DMA,
            pltpu.SemaphoreType.DMA,
        ],
    )
    return fn(xu, xi, edges_ub, edges_ib)


def _final_body(su, si, du, di, xu, xi, wo, bo, alu, aru, ali, ari,
                out_u, out_i):
    a_l_ub = alu[0]
    a_r_ub = aru[0]
    a_l_ib = ali[0]
    a_r_ib = ari[0]
    hi = a_l_ub * su[...].astype(jnp.float32) + (a_r_ub * du[...]) * xi[...]
    hi = jnp.where(hi > 0, hi, jnp.exp(hi) - 1.0)
    out_i[...] = hi
    hu = a_l_ib * si[...].astype(jnp.float32) + (a_r_ib * di[...]) * xu[...]
    hu = jnp.where(hu > 0, hu, jnp.exp(hu) - 1.0)
    out_u[...] = lax.dot_general(hu, wo[...], _DNT,
                                 preferred_element_type=jnp.float32) + bo[...]


def _finalize(s_ub, s_ib, deg_ub, deg_ib, xu, xi, wo, bo,
              alu, aru, ali, ari):
    return pl.pallas_call(
        _final_body,
        grid=(NRB,),
        in_specs=[
            pl.BlockSpec((ROWBLK, D), lambda i: (i, 0)),
            pl.BlockSpec((ROWBLK, D), lambda i: (i, 0)),
            pl.BlockSpec((ROWBLK, 1), lambda i: (i, 0)),
            pl.BlockSpec((ROWBLK, 1), lambda i: (i, 0)),
            pl.BlockSpec((ROWBLK, D), lambda i: (i, 0)),
            pl.BlockSpec((ROWBLK, D), lambda i: (i, 0)),
            pl.BlockSpec((D, D), lambda i: (0, 0)),
            pl.BlockSpec((1, D), lambda i: (0, 0)),
            pl.BlockSpec(memory_space=pltpu.SMEM),
            pl.BlockSpec(memory_space=pltpu.SMEM),
            pl.BlockSpec(memory_space=pltpu.SMEM),
            pl.BlockSpec(memory_space=pltpu.SMEM),
        ],
        out_specs=[
            pl.BlockSpec((ROWBLK, D), lambda i: (i, 0)),
            pl.BlockSpec((ROWBLK, D), lambda i: (i, 0)),
        ],
        out_shape=[jax.ShapeDtypeStruct((N, D), jnp.float32)] * 2,
    )(s_ub, s_ib, deg_ub, deg_ib, xu, xi, wo, bo, alu, aru, ali, ari)


def kernel(x_user, x_item, edge_index_ub, edge_index_ib, W_proj, b_proj,
           alpha_l_ub, alpha_r_ub, alpha_l_ib, alpha_r_ib, W_out, b_out):
    xu, xi, xu_bf, xi_bf = _project(x_user, x_item, W_proj,
                                    b_proj.reshape(1, D))

    pad = jnp.stack([jnp.zeros((E_PAD - E,), jnp.int32),
                     jnp.full((E_PAD - E,), N_PAD - 1, jnp.int32)])
    edges_ub = jnp.concatenate([edge_index_ub, pad], axis=1)
    edges_ib = jnp.concatenate([edge_index_ib, pad], axis=1)

    s_ub, deg_ub, s_ib, deg_ib = _sc_scatter(xu_bf, xi_bf, edges_ub, edges_ib)

    out_u, out_i = _finalize(
        s_ub, s_ib, deg_ub.reshape(N_PAD, 1), deg_ib.reshape(N_PAD, 1),
        xu, xi, W_out, b_out.reshape(1, D),
        alpha_l_ub, alpha_r_ub, alpha_l_ib, alpha_r_ib)
    return (out_u, out_i)


# speedup vs baseline: 6.8157x; 1.0134x over previous
import jax
import jax.numpy as jnp
from jax import lax
from jax.experimental import pallas as pl
from jax.experimental.pallas import tpu as pltpu
from jax.experimental.pallas import tpu_sc as plsc

N = 50000
D = 128
E = 400000

NC = 2
NS = 16
LANES = 16

ROWS_PER_TILE = 1568
R = NS * ROWS_PER_TILE
NPASS = 1
N_PAD = NC * R * NPASS
SEG = 1792
NSEG = 14
EPT = SEG * NSEG
E_PAD = EPT * NS
SBLK = SEG // LANES
CH = 128
DUMP = R
ROWBLK = 2000
NRB = N // ROWBLK


_DNT = (((1,), (1,)), ((), ()))


def _proj_body(xu_in, xi_in, wt, b, xu_out, xi_out, xu_bf_out, xi_bf_out):
    bb = b[...]
    xu = lax.dot_general(xu_in[...], wt[...], _DNT,
                         preferred_element_type=jnp.float32) + bb
    xi = lax.dot_general(xi_in[...], wt[...], _DNT,
                         preferred_element_type=jnp.float32) + bb
    xu_out[...] = xu
    xi_out[...] = xi
    xu_bf_out[...] = xu.astype(jnp.bfloat16)
    xi_bf_out[...] = xi.astype(jnp.bfloat16)


def _project(x_user, x_item, wt, b):
    return pl.pallas_call(
        _proj_body,
        grid=(NRB,),
        in_specs=[
            pl.BlockSpec((ROWBLK, D), lambda i: (i, 0)),
            pl.BlockSpec((ROWBLK, D), lambda i: (i, 0)),
            pl.BlockSpec((D, D), lambda i: (0, 0)),
            pl.BlockSpec((1, D), lambda i: (0, 0)),
        ],
        out_specs=[
            pl.BlockSpec((ROWBLK, D), lambda i: (i, 0)),
            pl.BlockSpec((ROWBLK, D), lambda i: (i, 0)),
            pl.BlockSpec((ROWBLK, D), lambda i: (i, 0)),
            pl.BlockSpec((ROWBLK, D), lambda i: (i, 0)),
        ],
        out_shape=[jax.ShapeDtypeStruct((N, D), jnp.float32)] * 2
        + [jax.ShapeDtypeStruct((N, D), jnp.bfloat16)] * 2,
    )(x_user, x_item, wt, b)


def _sc_body(xu, xi, edges_ub, edges_ib,
             s_ub_out, deg_ub_out, s_ib_out, deg_ib_out,
             seg_src0, seg_dst0, seg_src1, seg_dst1, gidx, sidx, rows0,
             rows1, ones_v, dz, deg_stage, accum, dega,
             sem0, sem1, semd, seme0, seme1, semr0, semr1):
    c = lax.axis_index("c")
    s = lax.axis_index("s")
    iota = lax.iota(jnp.int32, LANES)
    z16f = jnp.zeros((LANES,), jnp.float32)
    z32bf = jnp.zeros((2 * LANES,), jnp.bfloat16)
    dumpg16 = jnp.full((LANES,), s * ROWS_PER_TILE, jnp.int32)
    dump16 = jnp.full((LANES,), DUMP, jnp.int32)

    for cb in range(CH // LANES):
        ones_v[pl.ds(cb * LANES, LANES)] = jnp.ones((LANES,), jnp.float32)
    for cb in range(CH // LANES):
        dz[pl.ds(cb * LANES, LANES)] = z16f

    for (edges, table, s_out, d_out) in (
            (edges_ub, xu, s_ub_out, deg_ub_out),
            (edges_ib, xi, s_ib_out, deg_ib_out)):
        for p in range(NPASS):
            base = (p * NC + c) * R
            row00 = s * ROWS_PER_TILE

            for r_ in range(CH):
                for cb in range(D // (2 * LANES)):
                    rows0[r_, pl.ds(cb * 2 * LANES, 2 * LANES)] = z32bf
            for z in range(ROWS_PER_TILE // CH):
                pltpu.sync_copy(rows0, accum.at[pl.ds(row00 + z * CH, CH)])
            pltpu.sync_copy(
                rows0.at[pl.ds(0, ROWS_PER_TILE % CH)],
                accum.at[pl.ds(row00 + (ROWS_PER_TILE // CH) * CH,
                               ROWS_PER_TILE % CH)])
            for z in range(ROWS_PER_TILE // CH):
                pltpu.sync_copy(dz, dega.at[pl.ds(row00 + z * CH, CH)])
            pltpu.sync_copy(
                dz.at[pl.ds(0, ROWS_PER_TILE % CH)],
                dega.at[pl.ds(row00 + (ROWS_PER_TILE // CH) * CH,
                              ROWS_PER_TILE % CH)])

            @pl.when(s == 0)
            def _zero_dump():
                pltpu.sync_copy(rows0.at[pl.ds(0, 16)],
                                accum.at[pl.ds(R, 16)])
                pltpu.sync_copy(dz.at[pl.ds(0, 16)], dega.at[pl.ds(R, 16)])

            plsc.subcore_barrier()

            def fire_seg(g, ssb, sdb, sme):
                e0 = s * EPT + g * SEG
                pltpu.async_copy(edges.at[0, pl.ds(e0, SEG)], ssb, sme)
                pltpu.async_copy(edges.at[1, pl.ds(e0, SEG)], sdb, sme)

            def drain_seg(g, ssb, sdb, sme):
                e0 = s * EPT + g * SEG
                pltpu.make_async_copy(
                    edges.at[0, pl.ds(e0, SEG)], ssb, sme).wait()
                pltpu.make_async_copy(
                    edges.at[1, pl.ds(e0, SEG)], sdb, sme).wait()

            def segment(g, seg_src, seg_dst):
                def scan_blk(i, cnt):
                    sl = pl.ds(i * LANES, LANES)
                    d16 = seg_dst[sl]
                    s16 = seg_src[sl]
                    off = d16 - base
                    m = (off >= 0) & (off < R)
                    csum = plsc.cumsum(m.astype(jnp.int32))
                    pos = cnt - 1 + csum
                    plsc.store_scatter(gidx, [pos], s16, mask=m)
                    plsc.store_scatter(sidx, [pos], off, mask=m)
                    return cnt + jnp.sum(m.astype(jnp.int32))

                cnt = lax.fori_loop(0, SBLK, scan_blk, jnp.int32(0))

                nch = lax.shift_right_logical(cnt + (CH - 1), 7)

                def fill_blk(j, carry2):
                    pos = j * LANES + iota
                    m = pos >= cnt
                    plsc.store_scatter(gidx, [pos], dumpg16, mask=m)
                    plsc.store_scatter(sidx, [pos], dump16, mask=m)
                    return carry2

                lax.fori_loop(lax.shift_right_logical(cnt, 4),
                              nch * (CH // LANES), fill_blk, jnp.int32(0))

                def _fire(cc, buf, sem):
                    pltpu.async_copy(
                        table.at[gidx.at[pl.ds(cc * CH, CH)]], buf, sem)

                def _drain(cc, buf, sem):
                    pltpu.make_async_copy(
                        table.at[gidx.at[pl.ds(cc * CH, CH)]], buf, sem).wait()

                def _scat(cc, buf, semr):
                    ssl = sidx.at[pl.ds(cc * CH, CH)]
                    pltpu.async_copy(buf, accum.at[ssl], semr, add=True)
                    pltpu.async_copy(ones_v, dega.at[ssl], semd, add=True)

                def _drain_scat(buf, semr):
                    pltpu.make_async_copy(
                        buf, accum.at[pl.ds(0, CH)], semr).wait()

                @pl.when(nch > 0)
                def _pro0():
                    _fire(jnp.int32(0), rows0, sem0)

                @pl.when(nch > 1)
                def _pro1():
                    _fire(jnp.int32(1), rows1, sem1)

                def pair(pp, carry3):
                    c0 = pp * 2
                    _drain(c0, rows0, sem0)
                    _scat(c0, rows0, semr0)

                    @pl.when(c0 + 1 < nch)
                    def _h1():
                        _drain(c0 + 1, rows1, sem1)
                        _scat(c0 + 1, rows1, semr1)

                    @pl.when(c0 + 2 < nch)
                    def _f2():
                        _drain_scat(rows0, semr0)
                        _fire(c0 + 2, rows0, sem0)

                    @pl.when(c0 + 3 < nch)
                    def _f3():
                        _drain_scat(rows1, semr1)
                        _fire(c0 + 3, rows1, sem1)

                    return carry3

                lax.fori_loop(0, lax.shift_right_logical(nch + 1, 1), pair,
                              jnp.int32(0))

                odd = lax.rem(nch, 2)

                @pl.when((nch > 0) & (odd == 1))
                def _t0():
                    _drain_scat(rows0, semr0)

                @pl.when((nch > 0) & (odd == 0))
                def _t1():
                    _drain_scat(rows1, semr1)

                @pl.when((nch > 1) & (odd == 0))
                def _t2():
                    _drain_scat(rows0, semr0)

                @pl.when((nch > 1) & (odd == 1))
                def _t3():
                    _drain_scat(rows1, semr1)

                def ddrain(cc, carry4):
                    pltpu.make_async_copy(
                        ones_v, dega.at[pl.ds(0, CH)], semd).wait()
                    return carry4

                lax.fori_loop(0, nch, ddrain, jnp.int32(0))

            fire_seg(0, seg_src0, seg_dst0, seme0)
            fire_seg(1, seg_src1, seg_dst1, seme1)

            def segpair(q, carry):
                g0 = q * 2
                drain_seg(g0, seg_src0, seg_dst0, seme0)
                segment(g0, seg_src0, seg_dst0)

                @pl.when(g0 + 2 < NSEG)
                def _pf0():
                    fire_seg(g0 + 2, seg_src0, seg_dst0, seme0)

                drain_seg(g0 + 1, seg_src1, seg_dst1, seme1)
                segment(g0 + 1, seg_src1, seg_dst1)

                @pl.when(g0 + 3 < NSEG)
                def _pf1():
                    fire_seg(g0 + 3, seg_src1, seg_dst1, seme1)

                return carry

            lax.fori_loop(0, NSEG // 2, segpair, jnp.int32(0))
            plsc.subcore_barrier()

            row0 = s * ROWS_PER_TILE
            g0 = base + row0
            pltpu.sync_copy(accum.at[pl.ds(row0, ROWS_PER_TILE)],
                            s_out.at[pl.ds(g0, ROWS_PER_TILE)])
            pltpu.sync_copy(dega.at[pl.ds(row0, ROWS_PER_TILE)], deg_stage)
            pltpu.sync_copy(deg_stage, d_out.at[pl.ds(g0, ROWS_PER_TILE)])
            plsc.subcore_barrier()


def _sc_scatter(xu, xi, edges_ub, edges_ib):
    fn = pl.kernel(
        _sc_body,
        out_type=[
            jax.ShapeDtypeStruct((N_PAD, D), jnp.bfloat16),
            jax.ShapeDtypeStruct((N_PAD,), jnp.float32),
            jax.ShapeDtypeStruct((N_PAD, D), jnp.bfloat16),
            jax.ShapeDtypeStruct((N_PAD,), jnp.float32),
        ],
        mesh=plsc.VectorSubcoreMesh(core_axis_name="c", subcore_axis_name="s"),
        compiler_params=pltpu.CompilerParams(needs_layout_passes=False, use_tc_tiling_on_sc=False),
        scratch_types=[
            pltpu.VMEM((SEG,), jnp.int32),
            pltpu.VMEM((SEG,), jnp.int32),
            pltpu.VMEM((SEG,), jnp.int32),
            pltpu.VMEM((SEG,), jnp.int32),
            pltpu.VMEM((SEG,), jnp.int32),
            pltpu.VMEM((SEG,), jnp.int32),
            pltpu.VMEM((CH, D), jnp.bfloat16),
            pltpu.VMEM((CH, D), jnp.bfloat16),
            pltpu.VMEM((CH,), jnp.float32),
            pltpu.VMEM((CH,), jnp.float32),
            pltpu.VMEM((ROWS_PER_TILE,), jnp.float32),
            pltpu.VMEM_SHARED((R + 16, D), jnp.bfloat16),
            pltpu.VMEM_SHARED((R + 16,), jnp.float32),
            pltpu.SemaphoreType.DMA,
            pltpu.SemaphoreType.DMA,
            pltpu.SemaphoreType.DMA,
            pltpu.SemaphoreType.DMA,
            pltpu.SemaphoreType.DMA,
            pltpu.SemaphoreType.DMA,
            pltpu.SemaphoreType.DMA,
        ],
    )
    return fn(xu, xi, edges_ub, edges_ib)


def _final_body(su, si, du, di, xu, xi, wo, bo, alu, aru, ali, ari,
                out_u, out_i):
    a_l_ub = alu[0]
    a_r_ub = aru[0]
    a_l_ib = ali[0]
    a_r_ib = ari[0]
    hi = a_l_ub * su[...].astype(jnp.float32) + (a_r_ub * du[...]) * xi[...]
    hi = jnp.where(hi > 0, hi, jnp.exp(hi) - 1.0)
    out_i[...] = hi
    hu = a_l_ib * si[...].astype(jnp.float32) + (a_r_ib * di[...]) * xu[...]
    hu = jnp.where(hu > 0, hu, jnp.exp(hu) - 1.0)
    out_u[...] = lax.dot_general(hu, wo[...], _DNT,
                                 preferred_element_type=jnp.float32) + bo[...]


def _finalize(s_ub, s_ib, deg_ub, deg_ib, xu, xi, wo, bo,
              alu, aru, ali, ari):
    return pl.pallas_call(
        _final_body,
        grid=(NRB,),
        in_specs=[
            pl.BlockSpec((ROWBLK, D), lambda i: (i, 0)),
            pl.BlockSpec((ROWBLK, D), lambda i: (i, 0)),
            pl.BlockSpec((ROWBLK, 1), lambda i: (i, 0)),
            pl.BlockSpec((ROWBLK, 1), lambda i: (i, 0)),
            pl.BlockSpec((ROWBLK, D), lambda i: (i, 0)),
            pl.BlockSpec((ROWBLK, D), lambda i: (i, 0)),
            pl.BlockSpec((D, D), lambda i: (0, 0)),
            pl.BlockSpec((1, D), lambda i: (0, 0)),
            pl.BlockSpec(memory_space=pltpu.SMEM),
            pl.BlockSpec(memory_space=pltpu.SMEM),
            pl.BlockSpec(memory_space=pltpu.SMEM),
            pl.BlockSpec(memory_space=pltpu.SMEM),
        ],
        out_specs=[
            pl.BlockSpec((ROWBLK, D), lambda i: (i, 0)),
            pl.BlockSpec((ROWBLK, D), lambda i: (i, 0)),
        ],
        out_shape=[jax.ShapeDtypeStruct((N, D), jnp.float32)] * 2,
    )(s_ub, s_ib, deg_ub, deg_ib, xu, xi, wo, bo, alu, aru, ali, ari)


def kernel(x_user, x_item, edge_index_ub, edge_index_ib, W_proj, b_proj,
           alpha_l_ub, alpha_r_ub, alpha_l_ib, alpha_r_ib, W_out, b_out):
    xu, xi, xu_bf, xi_bf = _project(x_user, x_item, W_proj,
                                    b_proj.reshape(1, D))

    pad = jnp.stack([jnp.zeros((E_PAD - E,), jnp.int32),
                     jnp.full((E_PAD - E,), N_PAD - 1, jnp.int32)])
    edges_ub = jnp.concatenate([edge_index_ub, pad], axis=1)
    edges_ib = jnp.concatenate([edge_index_ib, pad], axis=1)

    s_ub, deg_ub, s_ib, deg_ib = _sc_scatter(xu_bf, xi_bf, edges_ub, edges_ib)

    out_u, out_i = _finalize(
        s_ub, s_ib, deg_ub.reshape(N_PAD, 1), deg_ib.reshape(N_PAD, 1),
        xu, xi, W_out, b_out.reshape(1, D),
        alpha_l_ub, alpha_r_ub, alpha_l_ib, alpha_r_ib)
    return (out_u, out_i)


# speedup vs baseline: 6.9460x; 1.0191x over previous
import jax
import jax.numpy as jnp
from jax import lax
from jax.experimental import pallas as pl
from jax.experimental.pallas import tpu as pltpu
from jax.experimental.pallas import tpu_sc as plsc

N = 50000
D = 128
E = 400000

NC = 2
NS = 16
LANES = 16

ROWS_PER_TILE = 1568
R = NS * ROWS_PER_TILE
NPASS = 1
N_PAD = NC * R * NPASS
SEG = 1792
NSEG = 14
EPT = SEG * NSEG
E_PAD = EPT * NS
SBLK = SEG // LANES
CH = 128
DUMP = R
ROWBLK = 2000
NRB = N // ROWBLK


_DNT = (((1,), (1,)), ((), ()))


def _proj_body(xu_in, xi_in, wt, b, xu_bf_out, xi_bf_out):
    bb = b[...]
    xu = lax.dot_general(xu_in[...], wt[...], _DNT,
                         preferred_element_type=jnp.float32) + bb
    xi = lax.dot_general(xi_in[...], wt[...], _DNT,
                         preferred_element_type=jnp.float32) + bb
    xu_bf_out[...] = xu.astype(jnp.bfloat16)
    xi_bf_out[...] = xi.astype(jnp.bfloat16)


def _project(x_user, x_item, wt, b):
    return pl.pallas_call(
        _proj_body,
        grid=(NRB,),
        in_specs=[
            pl.BlockSpec((ROWBLK, D), lambda i: (i, 0)),
            pl.BlockSpec((ROWBLK, D), lambda i: (i, 0)),
            pl.BlockSpec((D, D), lambda i: (0, 0)),
            pl.BlockSpec((1, D), lambda i: (0, 0)),
        ],
        out_specs=[
            pl.BlockSpec((ROWBLK, D), lambda i: (i, 0)),
            pl.BlockSpec((ROWBLK, D), lambda i: (i, 0)),
        ],
        out_shape=[jax.ShapeDtypeStruct((N, D), jnp.bfloat16)] * 2,
    )(x_user, x_item, wt, b)


def _sc_body(xu, xi, edges_ub, edges_ib,
             s_ub_out, deg_ub_out, s_ib_out, deg_ib_out,
             seg_src0, seg_dst0, seg_src1, seg_dst1, gidx, sidx, rows0,
             rows1, ones_v, dz, deg_stage, accum, dega,
             sem0, sem1, semd, seme0, seme1, semr0, semr1):
    c = lax.axis_index("c")
    s = lax.axis_index("s")
    iota = lax.iota(jnp.int32, LANES)
    z16f = jnp.zeros((LANES,), jnp.float32)
    z32bf = jnp.zeros((2 * LANES,), jnp.bfloat16)
    dumpg16 = jnp.full((LANES,), s * ROWS_PER_TILE, jnp.int32)
    dump16 = jnp.full((LANES,), DUMP, jnp.int32)

    for cb in range(CH // LANES):
        ones_v[pl.ds(cb * LANES, LANES)] = jnp.ones((LANES,), jnp.float32)
    for cb in range(CH // LANES):
        dz[pl.ds(cb * LANES, LANES)] = z16f

    for (edges, table, s_out, d_out) in (
            (edges_ub, xu, s_ub_out, deg_ub_out),
            (edges_ib, xi, s_ib_out, deg_ib_out)):
        for p in range(NPASS):
            base = (p * NC + c) * R
            row00 = s * ROWS_PER_TILE

            for r_ in range(CH):
                for cb in range(D // (2 * LANES)):
                    rows0[r_, pl.ds(cb * 2 * LANES, 2 * LANES)] = z32bf
            for z in range(ROWS_PER_TILE // CH):
                pltpu.sync_copy(rows0, accum.at[pl.ds(row00 + z * CH, CH)])
            pltpu.sync_copy(
                rows0.at[pl.ds(0, ROWS_PER_TILE % CH)],
                accum.at[pl.ds(row00 + (ROWS_PER_TILE // CH) * CH,
                               ROWS_PER_TILE % CH)])
            for z in range(ROWS_PER_TILE // CH):
                pltpu.sync_copy(dz, dega.at[pl.ds(row00 + z * CH, CH)])
            pltpu.sync_copy(
                dz.at[pl.ds(0, ROWS_PER_TILE % CH)],
                dega.at[pl.ds(row00 + (ROWS_PER_TILE // CH) * CH,
                              ROWS_PER_TILE % CH)])

            @pl.when(s == 0)
            def _zero_dump():
                pltpu.sync_copy(rows0.at[pl.ds(0, 16)],
                                accum.at[pl.ds(R, 16)])
                pltpu.sync_copy(dz.at[pl.ds(0, 16)], dega.at[pl.ds(R, 16)])

            plsc.subcore_barrier()

            def fire_seg(g, ssb, sdb, sme):
                e0 = s * EPT + g * SEG
                pltpu.async_copy(edges.at[0, pl.ds(e0, SEG)], ssb, sme)
                pltpu.async_copy(edges.at[1, pl.ds(e0, SEG)], sdb, sme)

            def drain_seg(g, ssb, sdb, sme):
                e0 = s * EPT + g * SEG
                pltpu.make_async_copy(
                    edges.at[0, pl.ds(e0, SEG)], ssb, sme).wait()
                pltpu.make_async_copy(
                    edges.at[1, pl.ds(e0, SEG)], sdb, sme).wait()

            def segment(g, seg_src, seg_dst):
                def scan_blk(i, cnt):
                    sl = pl.ds(i * LANES, LANES)
                    d16 = seg_dst[sl]
                    s16 = seg_src[sl]
                    off = d16 - base
                    m = (off >= 0) & (off < R)
                    csum = plsc.cumsum(m.astype(jnp.int32))
                    pos = cnt - 1 + csum
                    plsc.store_scatter(gidx, [pos], s16, mask=m)
                    plsc.store_scatter(sidx, [pos], off, mask=m)
                    return cnt + jnp.sum(m.astype(jnp.int32))

                cnt = lax.fori_loop(0, SBLK, scan_blk, jnp.int32(0))

                nch = lax.shift_right_logical(cnt + (CH - 1), 7)

                def fill_blk(j, carry2):
                    pos = j * LANES + iota
                    m = pos >= cnt
                    plsc.store_scatter(gidx, [pos], dumpg16, mask=m)
                    plsc.store_scatter(sidx, [pos], dump16, mask=m)
                    return carry2

                lax.fori_loop(lax.shift_right_logical(cnt, 4),
                              nch * (CH // LANES), fill_blk, jnp.int32(0))

                def _fire(cc, buf, sem):
                    pltpu.async_copy(
                        table.at[gidx.at[pl.ds(cc * CH, CH)]], buf, sem)

                def _drain(cc, buf, sem):
                    pltpu.make_async_copy(
                        table.at[gidx.at[pl.ds(cc * CH, CH)]], buf, sem).wait()

                def _scat(cc, buf, semr):
                    ssl = sidx.at[pl.ds(cc * CH, CH)]
                    pltpu.async_copy(buf, accum.at[ssl], semr, add=True)
                    pltpu.async_copy(ones_v, dega.at[ssl], semd, add=True)

                def _drain_scat(buf, semr):
                    pltpu.make_async_copy(
                        buf, accum.at[pl.ds(0, CH)], semr).wait()

                @pl.when(nch > 0)
                def _pro0():
                    _fire(jnp.int32(0), rows0, sem0)

                @pl.when(nch > 1)
                def _pro1():
                    _fire(jnp.int32(1), rows1, sem1)

                def pair(pp, carry3):
                    c0 = pp * 2
                    _drain(c0, rows0, sem0)
                    _scat(c0, rows0, semr0)

                    @pl.when(c0 + 1 < nch)
                    def _h1():
                        _drain(c0 + 1, rows1, sem1)
                        _scat(c0 + 1, rows1, semr1)

                    @pl.when(c0 + 2 < nch)
                    def _f2():
                        _drain_scat(rows0, semr0)
                        _fire(c0 + 2, rows0, sem0)

                    @pl.when(c0 + 3 < nch)
                    def _f3():
                        _drain_scat(rows1, semr1)
                        _fire(c0 + 3, rows1, sem1)

                    return carry3

                lax.fori_loop(0, lax.shift_right_logical(nch + 1, 1), pair,
                              jnp.int32(0))

                odd = lax.rem(nch, 2)

                @pl.when((nch > 0) & (odd == 1))
                def _t0():
                    _drain_scat(rows0, semr0)

                @pl.when((nch > 0) & (odd == 0))
                def _t1():
                    _drain_scat(rows1, semr1)

                @pl.when((nch > 1) & (odd == 0))
                def _t2():
                    _drain_scat(rows0, semr0)

                @pl.when((nch > 1) & (odd == 1))
                def _t3():
                    _drain_scat(rows1, semr1)

                def ddrain(cc, carry4):
                    pltpu.make_async_copy(
                        ones_v, dega.at[pl.ds(0, CH)], semd).wait()
                    return carry4

                lax.fori_loop(0, nch, ddrain, jnp.int32(0))

            fire_seg(0, seg_src0, seg_dst0, seme0)
            fire_seg(1, seg_src1, seg_dst1, seme1)

            def segpair(q, carry):
                g0 = q * 2
                drain_seg(g0, seg_src0, seg_dst0, seme0)
                segment(g0, seg_src0, seg_dst0)

                @pl.when(g0 + 2 < NSEG)
                def _pf0():
                    fire_seg(g0 + 2, seg_src0, seg_dst0, seme0)

                drain_seg(g0 + 1, seg_src1, seg_dst1, seme1)
                segment(g0 + 1, seg_src1, seg_dst1)

                @pl.when(g0 + 3 < NSEG)
                def _pf1():
                    fire_seg(g0 + 3, seg_src1, seg_dst1, seme1)

                return carry

            lax.fori_loop(0, NSEG // 2, segpair, jnp.int32(0))
            plsc.subcore_barrier()

            row0 = s * ROWS_PER_TILE
            g0 = base + row0
            pltpu.sync_copy(accum.at[pl.ds(row0, ROWS_PER_TILE)],
                            s_out.at[pl.ds(g0, ROWS_PER_TILE)])
            pltpu.sync_copy(dega.at[pl.ds(row0, ROWS_PER_TILE)], deg_stage)
            pltpu.sync_copy(deg_stage, d_out.at[pl.ds(g0, ROWS_PER_TILE)])
            plsc.subcore_barrier()


def _sc_scatter(xu, xi, edges_ub, edges_ib):
    fn = pl.kernel(
        _sc_body,
        out_type=[
            jax.ShapeDtypeStruct((N_PAD, D), jnp.bfloat16),
            jax.ShapeDtypeStruct((N_PAD,), jnp.float32),
            jax.ShapeDtypeStruct((N_PAD, D), jnp.bfloat16),
            jax.ShapeDtypeStruct((N_PAD,), jnp.float32),
        ],
        mesh=plsc.VectorSubcoreMesh(core_axis_name="c", subcore_axis_name="s"),
        compiler_params=pltpu.CompilerParams(needs_layout_passes=False, use_tc_tiling_on_sc=False),
        scratch_types=[
            pltpu.VMEM((SEG,), jnp.int32),
            pltpu.VMEM((SEG,), jnp.int32),
            pltpu.VMEM((SEG,), jnp.int32),
            pltpu.VMEM((SEG,), jnp.int32),
            pltpu.VMEM((SEG,), jnp.int32),
            pltpu.VMEM((SEG,), jnp.int32),
            pltpu.VMEM((CH, D), jnp.bfloat16),
            pltpu.VMEM((CH, D), jnp.bfloat16),
            pltpu.VMEM((CH,), jnp.float32),
            pltpu.VMEM((CH,), jnp.float32),
            pltpu.VMEM((ROWS_PER_TILE,), jnp.float32),
            pltpu.VMEM_SHARED((R + 16, D), jnp.bfloat16),
            pltpu.VMEM_SHARED((R + 16,), jnp.float32),
            pltpu.SemaphoreType.DMA,
            pltpu.SemaphoreType.DMA,
            pltpu.SemaphoreType.DMA,
            pltpu.SemaphoreType.DMA,
            pltpu.SemaphoreType.DMA,
            pltpu.SemaphoreType.DMA,
            pltpu.SemaphoreType.DMA,
        ],
    )
    return fn(xu, xi, edges_ub, edges_ib)


def _final_body(su, si, du, di, xur, xir, wp, bp, wo, bo,
                alu, aru, ali, ari, out_u, out_i):
    a_l_ub = alu[0]
    a_r_ub = aru[0]
    a_l_ib = ali[0]
    a_r_ib = ari[0]
    bb = bp[...]
    xu = lax.dot_general(xur[...], wp[...], _DNT,
                         preferred_element_type=jnp.float32) + bb
    xi = lax.dot_general(xir[...], wp[...], _DNT,
                         preferred_element_type=jnp.float32) + bb
    hi = a_l_ub * su[...].astype(jnp.float32) + (a_r_ub * du[...]) * xi
    hi = jnp.where(hi > 0, hi, jnp.exp(hi) - 1.0)
    out_i[...] = hi
    hu = a_l_ib * si[...].astype(jnp.float32) + (a_r_ib * di[...]) * xu
    hu = jnp.where(hu > 0, hu, jnp.exp(hu) - 1.0)
    out_u[...] = lax.dot_general(hu, wo[...], _DNT,
                                 preferred_element_type=jnp.float32) + bo[...]


def _finalize(s_ub, s_ib, deg_ub, deg_ib, xur, xir, wp, bp, wo, bo,
              alu, aru, ali, ari):
    return pl.pallas_call(
        _final_body,
        grid=(NRB,),
        in_specs=[
            pl.BlockSpec((ROWBLK, D), lambda i: (i, 0)),
            pl.BlockSpec((ROWBLK, D), lambda i: (i, 0)),
            pl.BlockSpec((ROWBLK, 1), lambda i: (i, 0)),
            pl.BlockSpec((ROWBLK, 1), lambda i: (i, 0)),
            pl.BlockSpec((ROWBLK, D), lambda i: (i, 0)),
            pl.BlockSpec((ROWBLK, D), lambda i: (i, 0)),
            pl.BlockSpec((D, D), lambda i: (0, 0)),
            pl.BlockSpec((1, D), lambda i: (0, 0)),
            pl.BlockSpec((D, D), lambda i: (0, 0)),
            pl.BlockSpec((1, D), lambda i: (0, 0)),
            pl.BlockSpec(memory_space=pltpu.SMEM),
            pl.BlockSpec(memory_space=pltpu.SMEM),
            pl.BlockSpec(memory_space=pltpu.SMEM),
            pl.BlockSpec(memory_space=pltpu.SMEM),
        ],
        out_specs=[
            pl.BlockSpec((ROWBLK, D), lambda i: (i, 0)),
            pl.BlockSpec((ROWBLK, D), lambda i: (i, 0)),
        ],
        out_shape=[jax.ShapeDtypeStruct((N, D), jnp.float32)] * 2,
    )(s_ub, s_ib, deg_ub, deg_ib, xur, xir, wp, bp, wo, bo,
      alu, aru, ali, ari)


def kernel(x_user, x_item, edge_index_ub, edge_index_ib, W_proj, b_proj,
           alpha_l_ub, alpha_r_ub, alpha_l_ib, alpha_r_ib, W_out, b_out):
    xu_bf, xi_bf = _project(x_user, x_item, W_proj, b_proj.reshape(1, D))

    pad = jnp.stack([jnp.zeros((E_PAD - E,), jnp.int32),
                     jnp.full((E_PAD - E,), N_PAD - 1, jnp.int32)])
    edges_ub = jnp.concatenate([edge_index_ub, pad], axis=1)
    edges_ib = jnp.concatenate([edge_index_ib, pad], axis=1)

    s_ub, deg_ub, s_ib, deg_ib = _sc_scatter(xu_bf, xi_bf, edges_ub, edges_ib)

    out_u, out_i = _finalize(
        s_ub, s_ib, deg_ub.reshape(N_PAD, 1), deg_ib.reshape(N_PAD, 1),
        x_user, x_item, W_proj, b_proj.reshape(1, D), W_out,
        b_out.reshape(1, D), alpha_l_ub, alpha_r_ub, alpha_l_ib, alpha_r_ib)
    return (out_u, out_i)
